# Initial kernel scaffold; baseline (speedup 1.0000x reference)
import jax
import jax.numpy as jnp
from jax.experimental import pallas as pl


def kernel(expressions_encodings, symbols_encodings, expr_idx, token_idx, symbol_idx, W_z, b_z, W_h, b_h):
    raise NotImplementedError("write your pallas kernel here")


# speedup vs baseline: 3.2909x; 3.2909x over previous
import functools

import jax
import jax.numpy as jnp
from jax.experimental import pallas as pl
from jax.experimental.pallas import tpu as pltpu


def _fused_body(flat_ref, u_ref, mask_ref, wtop_ref, wbot_ref, bias_ref, out_ref):
    prev = flat_ref[...]
    u = u_ref[...]
    zh = (
        jnp.dot(prev, wtop_ref[...], preferred_element_type=jnp.float32)
        + jnp.dot(u, wbot_ref[...], preferred_element_type=jnp.float32)
        + bias_ref[...]
    )
    d = prev.shape[-1]
    z = jax.nn.sigmoid(zh[:, :d])
    h = jnp.tanh(zh[:, d:])
    blk = prev.shape[0]
    maskf = mask_ref[0, 0, :].reshape(blk, 1)
    out_ref[...] = prev + (maskf * z) * (h - prev)


def _fused_update(flat, u, maskf, w_top, w_bot, bias, blk=512):
    n, d = flat.shape
    grid = n // blk
    mask3 = maskf.reshape(grid, 1, blk)
    return pl.pallas_call(
        _fused_body,
        grid=(grid,),
        in_specs=[
            pl.BlockSpec((blk, d), lambda i: (i, 0)),
            pl.BlockSpec((blk, d), lambda i: (i, 0)),
            pl.BlockSpec((1, 1, blk), lambda i: (i, 0, 0)),
            pl.BlockSpec((d, 2 * d), lambda i: (0, 0)),
            pl.BlockSpec((d, 2 * d), lambda i: (0, 0)),
            pl.BlockSpec((1, 2 * d), lambda i: (0, 0)),
        ],
        out_specs=pl.BlockSpec((blk, d), lambda i: (i, 0)),
        out_shape=jax.ShapeDtypeStruct((n, d), jnp.float32),
    )(flat, u, mask3, w_top, w_bot, bias)


def kernel(expressions_encodings, symbols_encodings, expr_idx, token_idx, symbol_idx, W_z, b_z, W_h, b_h):
    b, l, d = expressions_encodings.shape
    n = b * l
    e = expr_idx.shape[0]
    flat_idx = l * expr_idx.astype(jnp.int32) + token_idx.astype(jnp.int32)
    flat = expressions_encodings.reshape(n, d)

    winner = jnp.full((n,), -1, jnp.int32).at[flat_idx].max(
        jnp.arange(e, dtype=jnp.int32), mode="drop"
    )
    win_sym = jnp.take(symbol_idx.astype(jnp.int32), jnp.maximum(winner, 0), axis=0)
    u = jnp.take(symbols_encodings, win_sym, axis=0)
    maskf = (winner >= 0).astype(jnp.float32)

    w_top = jnp.concatenate([W_z[:d], W_h[:d]], axis=1)
    w_bot = jnp.concatenate([W_z[d:], W_h[d:]], axis=1)
    bias = jnp.concatenate([b_z, b_h]).reshape(1, 2 * d)

    out = _fused_update(flat, u, maskf, w_top, w_bot, bias)
    return out.reshape(b, l, d)


# speedup vs baseline: 3.4972x; 1.0627x over previous
import functools

import jax
import jax.numpy as jnp
from jax import lax
from jax.experimental import pallas as pl
from jax.experimental.pallas import tpu as pltpu
from jax.experimental.pallas import tpu_sc as plsc

_NC = 2
_NS = 16
_NW = _NC * _NS
_LANES = 16


def _mesh():
    return plsc.VectorSubcoreMesh(
        core_axis_name="c", subcore_axis_name="s", num_cores=_NC, num_subcores=_NS
    )


def _winner_tables(flat_idx, symbol_idx, neg1, n_rows):
    e = flat_idx.shape[0]
    chunk = e // _NW

    @functools.partial(
        pl.kernel,
        out_type=jax.ShapeDtypeStruct((_NW, n_rows), jnp.int32),
        mesh=_mesh(),
        compiler_params=pltpu.CompilerParams(needs_layout_passes=False),
        scratch_types=[
            pltpu.VMEM((chunk,), jnp.int32),
            pltpu.VMEM((chunk,), jnp.int32),
            pltpu.VMEM((n_rows,), jnp.int32),
            pltpu.VMEM((2 * _LANES,), jnp.int32),
        ],
    )
    def k(fi_hbm, si_hbm, neg1_hbm, tables_hbm, fi_v, si_v, tbl_v, shf_v):
        wid = lax.axis_index("s") * _NC + lax.axis_index("c")
        base = wid * chunk
        pltpu.sync_copy(fi_hbm.at[pl.ds(base, chunk)], fi_v)
        pltpu.sync_copy(si_hbm.at[pl.ds(base, chunk)], si_v)
        pltpu.sync_copy(neg1_hbm, tbl_v)
        lanes = lax.iota(jnp.int32, _LANES)
        shf_v[pl.ds(_LANES, _LANES)] = jnp.zeros((_LANES,), jnp.int32)

        def body(g, carry):
            off = g * _LANES
            fi = fi_v[pl.ds(off, _LANES)]
            si = si_v[pl.ds(off, _LANES)]
            key = fi * chunk + (off + lanes)
            skey, sval = plsc.sort_key_val(key, si)
            row = skey // chunk
            shf_v[pl.ds(0, _LANES)] = row
            nxt = shf_v[pl.ds(1, _LANES)]
            surv = (row != nxt) | (lanes == _LANES - 1)
            plsc.store_scatter(tbl_v, [row], sval, mask=surv)
            return carry

        lax.fori_loop(0, chunk // _LANES, body, 0)
        pltpu.sync_copy(tbl_v, tables_hbm.at[wid])

    return k(flat_idx, symbol_idx, neg1)


def _combine_and_gather(tables, symbols, n_rows, d):
    rows_per_w = n_rows // _NW
    half = rows_per_w // 2
    gb = 64
    nb = rows_per_w // gb

    @functools.partial(
        pl.kernel,
        out_type=(
            jax.ShapeDtypeStruct((n_rows,), jnp.int32),
            jax.ShapeDtypeStruct((n_rows, d), jnp.float32),
        ),
        mesh=_mesh(),
        scratch_types=[
            pltpu.VMEM((_NW, half), jnp.int32),
            pltpu.VMEM((rows_per_w,), jnp.int32),
            pltpu.VMEM((rows_per_w,), jnp.int32),
            pltpu.VMEM((gb, d), jnp.float32),
            pltpu.SemaphoreType.DMA,
        ],
    )
    def k(tables_hbm, sym_hbm, win_hbm, u_hbm, tbuf, win_v, idx_v, gbuf, sem):
        wid = lax.axis_index("s") * _NC + lax.axis_index("c")
        rbase = wid * rows_per_w
        lanes_per_half = half // _LANES
        for h in range(2):
            for c in range(_NW):
                pltpu.sync_copy(
                    tables_hbm.at[c, pl.ds(rbase + h * half, half)], tbuf.at[c]
                )

            def body(g, carry, h=h):
                acc = jnp.full((_LANES,), -1, jnp.int32)
                for c in range(_NW):
                    t = tbuf[c, pl.ds(g * _LANES, _LANES)]
                    acc = jnp.where(t >= 0, t, acc)
                win_v[pl.ds(h * half + g * _LANES, _LANES)] = acc
                idx_v[pl.ds(h * half + g * _LANES, _LANES)] = jnp.maximum(acc, 0)
                return carry

            lax.fori_loop(0, lanes_per_half, body, 0)
        pltpu.sync_copy(win_v, win_hbm.at[pl.ds(rbase, rows_per_w)])

        def gbody(g, carry):
            pltpu.async_copy(
                sym_hbm.at[idx_v.at[pl.ds(g * gb, gb)]], gbuf, sem
            ).wait()
            pltpu.sync_copy(gbuf, u_hbm.at[pl.ds(rbase + g * gb, gb)])
            return carry

        lax.fori_loop(0, nb, gbody, 0)

    return k(tables, symbols)


def _fused_body(flat_ref, u_ref, win_ref, wtop_ref, wbot_ref, bias_ref, out_ref):
    prev = flat_ref[...]
    u = u_ref[...]
    zh = (
        jnp.dot(prev, wtop_ref[...], preferred_element_type=jnp.float32)
        + jnp.dot(u, wbot_ref[...], preferred_element_type=jnp.float32)
        + bias_ref[...]
    )
    d = prev.shape[-1]
    z = jax.nn.sigmoid(zh[:, :d])
    h = jnp.tanh(zh[:, d:])
    blk = prev.shape[0]
    maskf = (win_ref[0, 0, :] >= 0).astype(jnp.float32).reshape(blk, 1)
    out_ref[...] = prev + (maskf * z) * (h - prev)


def _fused_update(flat, u, win_sym, w_top, w_bot, bias, blk=512):
    n, d = flat.shape
    grid = n // blk
    win3 = win_sym.reshape(grid, 1, blk)
    return pl.pallas_call(
        _fused_body,
        grid=(grid,),
        in_specs=[
            pl.BlockSpec((blk, d), lambda i: (i, 0)),
            pl.BlockSpec((blk, d), lambda i: (i, 0)),
            pl.BlockSpec((1, 1, blk), lambda i: (i, 0, 0)),
            pl.BlockSpec((d, 2 * d), lambda i: (0, 0)),
            pl.BlockSpec((d, 2 * d), lambda i: (0, 0)),
            pl.BlockSpec((1, 2 * d), lambda i: (0, 0)),
        ],
        out_specs=pl.BlockSpec((blk, d), lambda i: (i, 0)),
        out_shape=jax.ShapeDtypeStruct((n, d), jnp.float32),
    )(flat, u, win3, w_top, w_bot, bias)


def kernel(expressions_encodings, symbols_encodings, expr_idx, token_idx, symbol_idx, W_z, b_z, W_h, b_h):
    b, l, d = expressions_encodings.shape
    n = b * l
    flat_idx = l * expr_idx.astype(jnp.int32) + token_idx.astype(jnp.int32)
    flat = expressions_encodings.reshape(n, d)
    neg1 = jnp.full((n,), -1, jnp.int32)

    tables = _winner_tables(flat_idx, symbol_idx.astype(jnp.int32), neg1, n)
    win_sym, u = _combine_and_gather(tables, symbols_encodings, n, d)

    w_top = jnp.concatenate([W_z[:d], W_h[:d]], axis=1)
    w_bot = jnp.concatenate([W_z[d:], W_h[d:]], axis=1)
    bias = jnp.concatenate([b_z, b_h]).reshape(1, 2 * d)

    out = _fused_update(flat, u, win_sym, w_top, w_bot, bias)
    return out.reshape(b, l, d)


# speedup vs baseline: 3.6335x; 1.0390x over previous
import functools

import jax
import jax.numpy as jnp
from jax import lax
from jax.experimental import pallas as pl
from jax.experimental.pallas import tpu as pltpu
from jax.experimental.pallas import tpu_sc as plsc

_NC = 2
_NS = 16
_NW = _NC * _NS
_LANES = 16


def _mesh():
    return plsc.VectorSubcoreMesh(
        core_axis_name="c", subcore_axis_name="s", num_cores=_NC, num_subcores=_NS
    )


def _winner_tables(flat_idx, symbol_idx, neg1, n_rows):
    e = flat_idx.shape[0]
    chunk = e // _NW

    @functools.partial(
        pl.kernel,
        out_type=jax.ShapeDtypeStruct((_NW, n_rows), jnp.int32),
        mesh=_mesh(),
        compiler_params=pltpu.CompilerParams(needs_layout_passes=False),
        scratch_types=[
            pltpu.VMEM((chunk,), jnp.int32),
            pltpu.VMEM((chunk,), jnp.int32),
            pltpu.VMEM((n_rows,), jnp.int32),
            pltpu.VMEM((2 * _LANES,), jnp.int32),
        ],
    )
    def k(fi_hbm, si_hbm, neg1_hbm, tables_hbm, fi_v, si_v, tbl_v, shf_v):
        wid = lax.axis_index("s") * _NC + lax.axis_index("c")
        base = wid * chunk
        pltpu.sync_copy(fi_hbm.at[pl.ds(base, chunk)], fi_v)
        pltpu.sync_copy(si_hbm.at[pl.ds(base, chunk)], si_v)
        pltpu.sync_copy(neg1_hbm, tbl_v)
        lanes = lax.iota(jnp.int32, _LANES)
        shf_v[pl.ds(_LANES, _LANES)] = jnp.zeros((_LANES,), jnp.int32)

        def body(g, carry):
            off = g * _LANES
            fi = fi_v[pl.ds(off, _LANES)]
            si = si_v[pl.ds(off, _LANES)]
            key = fi * chunk + (off + lanes)
            skey, sval = plsc.sort_key_val(key, si)
            row = skey // chunk
            shf_v[pl.ds(0, _LANES)] = row
            nxt = shf_v[pl.ds(1, _LANES)]
            surv = (row != nxt) | (lanes == _LANES - 1)
            plsc.store_scatter(tbl_v, [row], sval, mask=surv)
            return carry

        lax.fori_loop(0, chunk // _LANES, body, 0)
        pltpu.sync_copy(tbl_v, tables_hbm.at[wid])

    return k(flat_idx, symbol_idx, neg1)


def _combine_and_gather(tables, symbols, n_rows, d):
    rows_per_w = n_rows // _NW
    half = rows_per_w // 2
    gb = 64
    nb = rows_per_w // gb

    @functools.partial(
        pl.kernel,
        out_type=(
            jax.ShapeDtypeStruct((n_rows,), jnp.int32),
            jax.ShapeDtypeStruct((n_rows, d), jnp.float32),
        ),
        mesh=_mesh(),
        scratch_types=[
            pltpu.VMEM((_NW, half), jnp.int32),
            pltpu.VMEM((rows_per_w,), jnp.int32),
            pltpu.VMEM((rows_per_w,), jnp.int32),
            pltpu.VMEM((gb, d), jnp.float32),
            pltpu.VMEM((gb, d), jnp.float32),
            pltpu.SemaphoreType.DMA,
            pltpu.SemaphoreType.DMA,
        ],
    )
    def k(tables_hbm, sym_hbm, win_hbm, u_hbm, tbuf, win_v, idx_v, gb0, gb1, gsem, osem):
        wid = lax.axis_index("s") * _NC + lax.axis_index("c")
        rbase = wid * rows_per_w
        lanes_per_half = half // _LANES
        for h in range(2):
            pltpu.sync_copy(tables_hbm.at[:, pl.ds(rbase + h * half, half)], tbuf)

            def body(g, carry, h=h):
                acc = jnp.full((_LANES,), -1, jnp.int32)
                for c in range(_NW):
                    t = tbuf[c, pl.ds(g * _LANES, _LANES)]
                    acc = jnp.where(t >= 0, t, acc)
                win_v[pl.ds(h * half + g * _LANES, _LANES)] = acc
                idx_v[pl.ds(h * half + g * _LANES, _LANES)] = jnp.maximum(acc, 0)
                return carry

            lax.fori_loop(0, lanes_per_half, body, 0)
        pltpu.sync_copy(win_v, win_hbm.at[pl.ds(rbase, rows_per_w)])

        def gstart(g, buf):
            pltpu.async_copy(sym_hbm.at[idx_v.at[pl.ds(g * gb, gb)]], buf, gsem)

        def gwait(buf):
            pltpu.make_async_copy(sym_hbm.at[idx_v.at[pl.ds(0, gb)]], buf, gsem).wait()

        def ostart(g, buf):
            pltpu.async_copy(buf, u_hbm.at[pl.ds(rbase + g * gb, gb)], osem)

        def owait(buf):
            pltpu.make_async_copy(buf, u_hbm.at[pl.ds(rbase, gb)], osem).wait()

        gstart(0, gb0)

        def pbody(i, carry):
            g = 2 * i

            @pl.when(i > 0)
            def _():
                owait(gb1)

            gwait(gb0)
            gstart(g + 1, gb1)
            ostart(g, gb0)
            gwait(gb1)
            owait(gb0)

            @pl.when(g + 2 < nb)
            def _():
                gstart(g + 2, gb0)

            ostart(g + 1, gb1)
            return carry

        lax.fori_loop(0, nb // 2, pbody, 0)
        owait(gb1)

    return k(tables, symbols)


def _fused_body(flat_ref, u_ref, win_ref, wtop_ref, wbot_ref, bias_ref, out_ref):
    prev = flat_ref[...]
    u = u_ref[...]
    zh = (
        jnp.dot(prev.astype(jnp.bfloat16), wtop_ref[...], preferred_element_type=jnp.float32)
        + jnp.dot(u.astype(jnp.bfloat16), wbot_ref[...], preferred_element_type=jnp.float32)
        + bias_ref[...]
    )
    d = prev.shape[-1]
    z = jax.nn.sigmoid(zh[:, :d])
    h = jnp.tanh(zh[:, d:])
    blk = prev.shape[0]
    maskf = (win_ref[0, 0, :] >= 0).astype(jnp.float32).reshape(blk, 1)
    out_ref[...] = prev + (maskf * z) * (h - prev)


def _fused_update(flat, u, win_sym, w_top, w_bot, bias, blk=512):
    n, d = flat.shape
    grid = n // blk
    win3 = win_sym.reshape(grid, 1, blk)
    return pl.pallas_call(
        _fused_body,
        grid=(grid,),
        in_specs=[
            pl.BlockSpec((blk, d), lambda i: (i, 0)),
            pl.BlockSpec((blk, d), lambda i: (i, 0)),
            pl.BlockSpec((1, 1, blk), lambda i: (i, 0, 0)),
            pl.BlockSpec((d, 2 * d), lambda i: (0, 0)),
            pl.BlockSpec((d, 2 * d), lambda i: (0, 0)),
            pl.BlockSpec((1, 2 * d), lambda i: (0, 0)),
        ],
        out_specs=pl.BlockSpec((blk, d), lambda i: (i, 0)),
        out_shape=jax.ShapeDtypeStruct((n, d), jnp.float32),
    )(flat, u, win3, w_top, w_bot, bias)


def kernel(expressions_encodings, symbols_encodings, expr_idx, token_idx, symbol_idx, W_z, b_z, W_h, b_h):
    b, l, d = expressions_encodings.shape
    n = b * l
    flat_idx = l * expr_idx.astype(jnp.int32) + token_idx.astype(jnp.int32)
    flat = expressions_encodings.reshape(n, d)
    neg1 = jnp.full((n,), -1, jnp.int32)

    tables = _winner_tables(flat_idx, symbol_idx.astype(jnp.int32), neg1, n)
    win_sym, u = _combine_and_gather(tables, symbols_encodings, n, d)

    w_top = jnp.concatenate([W_z[:d], W_h[:d]], axis=1).astype(jnp.bfloat16)
    w_bot = jnp.concatenate([W_z[d:], W_h[d:]], axis=1).astype(jnp.bfloat16)
    bias = jnp.concatenate([b_z, b_h]).reshape(1, 2 * d)

    out = _fused_update(flat, u, win_sym, w_top, w_bot, bias)
    return out.reshape(b, l, d)
